# baseline (speedup 1.0000x reference)
import functools

import jax
import jax.numpy as jnp
from jax import lax
from jax.experimental import pallas as pl
from jax.experimental.pallas import tpu as pltpu
from jax.experimental.pallas import tpu_sc as plsc

B = 16384
DIM = 32
NROWS = 1000000
NC = 2
NS = 16
NW = NC * NS
PW = B // NW
SLAB = 512
NPAD = ((NROWS + 127) // 128) * 128
MAXBASE = NPAD - SLAB


def _count_below(idxv, end):
    n = jnp.int32(0)
    for g in range(PW // 16):
        v = idxv[g]
        n = n + jnp.sum((v < end).astype(jnp.int32))
    return n


def _id_at(idxv, p, lanes):
    v = idxv[p // 16]
    return jnp.sum(v * (lanes == p % 16).astype(jnp.int32))


def _sc_scan_body(ids_hbm, pos_hbm, tt, out, idxv, pos_v, ring, buf,
                  sem0, sem1, sem_st):
    wid = lax.axis_index("s") * NC + lax.axis_index("c")
    pltpu.sync_copy(ids_hbm.at[pl.ds(wid * (PW // 16), PW // 16)], idxv)
    pltpu.sync_copy(pos_hbm.at[pl.ds(wid * (PW // 128), PW // 128)], pos_v)

    lanes = lax.iota(jnp.int32, 16)
    first = _id_at(idxv, jnp.int32(0), lanes)
    last = _id_at(idxv, jnp.int32(PW - 1), lanes)
    c_lo = jnp.minimum((first // 128) * 128, MAXBASE)
    n_slabs = (last - c_lo) // SLAB + 1
    n_pairs = (n_slabs + 1) // 2
    r0 = lax.iota(jnp.int32, 16)
    r1 = r0 + 16

    def slab_base(s):
        return pl.multiple_of(jnp.minimum(c_lo + s * SLAB, MAXBASE), 128)

    def start(s, slot, sem):
        pltpu.make_async_copy(
            tt.at[:, pl.ds(slab_base(s), SLAB)], ring.at[slot], sem).start()

    def wait(slot, sem):
        pltpu.make_async_copy(
            tt.at[:, pl.ds(0, SLAB)], ring.at[slot], sem).wait()

    def extract(s, slot, ptr):
        base = slab_base(s)
        nend = _count_below(idxv, base + SLAB)

        def ext(p, c):
            col = _id_at(idxv, p, lanes) - base
            cv = jnp.full((16,), col, jnp.int32)
            v0 = plsc.load_gather(ring.at[slot], [r0, cv])
            v1 = plsc.load_gather(ring.at[slot], [r1, cv])
            buf[p, pl.ds(0, 16)] = v0
            buf[p, pl.ds(16, 16)] = v1
            return c

        lax.fori_loop(ptr, nend, ext, jnp.int32(0))
        return nend

    start(jnp.int32(0), 0, sem0)

    def pair_step(k, ptr):
        s0 = 2 * k
        start(s0 + 1, 1, sem1)
        wait(0, sem0)
        ptr = extract(s0, 0, ptr)
        start(s0 + 2, 0, sem0)
        wait(1, sem1)
        return extract(s0 + 1, 1, ptr)

    lax.fori_loop(0, n_pairs, pair_step, jnp.int32(0))
    wait(0, sem0)

    sc = []
    for j in range(PW // 128):
        sc.append(pltpu.async_copy(
            buf.at[pl.ds(j * 128, 128)], out.at[pos_v.at[j]], sem_st))
    for c in sc:
        c.wait()


_sc_scan = functools.partial(
    pl.kernel,
    out_type=jax.ShapeDtypeStruct((B, 128), jnp.float32),
    mesh=plsc.VectorSubcoreMesh(core_axis_name="c", subcore_axis_name="s"),
    scratch_types=[
        pltpu.VMEM((PW // 16, 16), jnp.int32),
        pltpu.VMEM((PW // 128, 128), jnp.int32),
        pltpu.VMEM((2, DIM, SLAB), jnp.float32),
        pltpu.VMEM((PW, 128), jnp.float32),
        pltpu.SemaphoreType.DMA,
        pltpu.SemaphoreType.DMA,
        pltpu.SemaphoreType.DMA,
    ],
    compiler_params=pltpu.CompilerParams(needs_layout_passes=False),
)(_sc_scan_body)


def _tc_dense_body(gu_ref, gi_ref, wu_ref, bu_ref, wi_ref, bi_ref, out_ref):
    ue = gu_ref[...][:, :DIM]
    ie = gi_ref[...][:, :DIM]
    u = jnp.maximum(
        jnp.dot(ue, wu_ref[...], preferred_element_type=jnp.float32)
        + bu_ref[...], 0.0)
    v = jnp.maximum(
        jnp.dot(ie, wi_ref[...], preferred_element_type=jnp.float32)
        + bi_ref[...], 0.0)
    out_ref[...] = jnp.sum(u * v, axis=1, keepdims=True)


def _tc_dense(gu, gi, Wu, bu2, Wi, bi2):
    blk = 2048
    grid = B // blk
    return pl.pallas_call(
        _tc_dense_body,
        grid=(grid,),
        in_specs=[
            pl.BlockSpec((blk, 128), lambda i: (i, 0)),
            pl.BlockSpec((blk, 128), lambda i: (i, 0)),
            pl.BlockSpec((DIM, DIM), lambda i: (0, 0)),
            pl.BlockSpec((1, DIM), lambda i: (0, 0)),
            pl.BlockSpec((DIM, DIM), lambda i: (0, 0)),
            pl.BlockSpec((1, DIM), lambda i: (0, 0)),
        ],
        out_specs=pl.BlockSpec((blk, 1), lambda i: (i, 0)),
        out_shape=jax.ShapeDtypeStruct((B, 1), jnp.float32),
    )(gu, gi, Wu, bu2, Wi, bi2)


def kernel(user_table, item_table, Wu, bu, Wi, bi, user_ids, item_ids):
    order_u = jnp.argsort(user_ids).astype(jnp.int32)
    su = user_ids[order_u].reshape(B // 16, 16)
    pu_pos = order_u.reshape(B // 128, 128)
    gu = _sc_scan(su, pu_pos, user_table.T)

    order_i = jnp.argsort(item_ids).astype(jnp.int32)
    si = item_ids[order_i].reshape(B // 16, 16)
    pi_pos = order_i.reshape(B // 128, 128)
    gi = _sc_scan(si, pi_pos, item_table.T)

    scores = _tc_dense(gu, gi, Wu, bu.reshape(1, DIM), Wi, bi.reshape(1, DIM))
    return scores.reshape(B)


# speedup vs baseline: 1.0091x; 1.0091x over previous
import functools

import jax
import jax.numpy as jnp
from jax import lax
from jax.experimental import pallas as pl
from jax.experimental.pallas import tpu as pltpu
from jax.experimental.pallas import tpu_sc as plsc

B = 16384
DIM = 32
NROWS = 1000000
NC = 2
NS = 16
NW = NC * NS
PW = B // NW
SLAB = 512
NPAD = ((NROWS + 127) // 128) * 128
MAXBASE = NPAD - SLAB


def _count_below(idxv, end):
    n = jnp.int32(0)
    for g in range(PW // 16):
        v = idxv[g]
        n = n + jnp.sum((v < end).astype(jnp.int32))
    return n


def _id_at(idxv, p, lanes):
    v = idxv[p // 16]
    return jnp.sum(v * (lanes == p % 16).astype(jnp.int32))


def _sc_scan_body(ids_hbm, pos_hbm, tt, out, idxv, pos_v, ring, buf,
                  sem0, sem1, sem_st):
    wid = lax.axis_index("s") * NC + lax.axis_index("c")
    pltpu.sync_copy(ids_hbm.at[pl.ds(wid * (PW // 16), PW // 16)], idxv)
    pltpu.sync_copy(pos_hbm.at[pl.ds(wid * (PW // 128), PW // 128)], pos_v)

    lanes = lax.iota(jnp.int32, 16)
    first = _id_at(idxv, jnp.int32(0), lanes)
    last = _id_at(idxv, jnp.int32(PW - 1), lanes)
    c_lo = jnp.minimum((first // 128) * 128, MAXBASE)
    n_slabs = (last - c_lo) // SLAB + 1
    n_pairs = (n_slabs + 1) // 2
    r0 = lax.iota(jnp.int32, 16)
    r1 = r0 + 16

    def slab_base(s):
        return pl.multiple_of(jnp.minimum(c_lo + s * SLAB, MAXBASE), 128)

    def start(s, slot, sem):
        pltpu.make_async_copy(
            tt.at[:, pl.ds(slab_base(s), SLAB)], ring.at[slot], sem).start()

    def wait(slot, sem):
        pltpu.make_async_copy(
            tt.at[:, pl.ds(0, SLAB)], ring.at[slot], sem).wait()

    def extract(s, slot, ptr):
        base = slab_base(s)
        nend = _count_below(idxv, base + SLAB)

        def ext(p, c):
            col = _id_at(idxv, p, lanes) - base
            cv = jnp.full((16,), col, jnp.int32)
            v0 = plsc.load_gather(ring.at[slot], [r0, cv])
            v1 = plsc.load_gather(ring.at[slot], [r1, cv])
            buf[p, pl.ds(0, 16)] = v0
            buf[p, pl.ds(16, 16)] = v1
            return c

        lax.fori_loop(ptr, nend, ext, jnp.int32(0))
        return nend

    start(jnp.int32(0), 0, sem0)

    def pair_step(k, ptr):
        s0 = 2 * k
        start(s0 + 1, 1, sem1)
        wait(0, sem0)
        ptr = extract(s0, 0, ptr)
        start(s0 + 2, 0, sem0)
        wait(1, sem1)
        return extract(s0 + 1, 1, ptr)

    lax.fori_loop(0, n_pairs, pair_step, jnp.int32(0))
    wait(0, sem0)

    sc = []
    for j in range(PW // 128):
        sc.append(pltpu.async_copy(
            buf.at[pl.ds(j * 128, 128)], out.at[pos_v.at[j]], sem_st))
    for c in sc:
        c.wait()


_sc_scan = functools.partial(
    pl.kernel,
    out_type=jax.ShapeDtypeStruct((B, 128), jnp.float32),
    mesh=plsc.VectorSubcoreMesh(core_axis_name="c", subcore_axis_name="s"),
    scratch_types=[
        pltpu.VMEM((PW // 16, 16), jnp.int32),
        pltpu.VMEM((PW // 128, 128), jnp.int32),
        pltpu.VMEM((2, DIM, SLAB), jnp.float32),
        pltpu.VMEM((PW, 128), jnp.float32),
        pltpu.SemaphoreType.DMA,
        pltpu.SemaphoreType.DMA,
        pltpu.SemaphoreType.DMA,
    ],
    compiler_params=pltpu.CompilerParams(needs_layout_passes=False),
)(_sc_scan_body)


def _tc_dense_body(gu_ref, gi_ref, wu_ref, bu_ref, wi_ref, bi_ref, out_ref):
    ue = gu_ref[...][:, :DIM]
    ie = gi_ref[...][:, :DIM]
    u = jnp.maximum(
        jnp.dot(ue, wu_ref[...], preferred_element_type=jnp.float32)
        + bu_ref[...], 0.0)
    v = jnp.maximum(
        jnp.dot(ie, wi_ref[...], preferred_element_type=jnp.float32)
        + bi_ref[...], 0.0)
    out_ref[...] = jnp.sum(u * v, axis=1, keepdims=True)


def _tc_dense(gu, gi, Wu, bu2, Wi, bi2):
    blk = 2048
    grid = B // blk
    return pl.pallas_call(
        _tc_dense_body,
        grid=(grid,),
        in_specs=[
            pl.BlockSpec((blk, 128), lambda i: (i, 0)),
            pl.BlockSpec((blk, 128), lambda i: (i, 0)),
            pl.BlockSpec((DIM, DIM), lambda i: (0, 0)),
            pl.BlockSpec((1, DIM), lambda i: (0, 0)),
            pl.BlockSpec((DIM, DIM), lambda i: (0, 0)),
            pl.BlockSpec((1, DIM), lambda i: (0, 0)),
        ],
        out_specs=pl.BlockSpec((blk, 1), lambda i: (i, 0)),
        out_shape=jax.ShapeDtypeStruct((B, 1), jnp.float32),
    )(gu, gi, Wu, bu2, Wi, bi2)


def kernel(user_table, item_table, Wu, bu, Wi, bi, user_ids, item_ids):
    order_u = jnp.argsort(user_ids, stable=False).astype(jnp.int32)
    order_i = jnp.argsort(item_ids, stable=False).astype(jnp.int32)
    su = user_ids[order_u].reshape(B // 16, 16)
    si = item_ids[order_i].reshape(B // 16, 16)
    pu_pos = order_u.reshape(B // 128, 128)
    pi_pos = order_i.reshape(B // 128, 128)
    gu = _sc_scan(su, pu_pos, user_table.T)
    gi = _sc_scan(si, pi_pos, item_table.T)

    scores = _tc_dense(gu, gi, Wu, bu.reshape(1, DIM), Wi, bi.reshape(1, DIM))
    return scores.reshape(B)


# speedup vs baseline: 1.0639x; 1.0543x over previous
import functools

import jax
import jax.numpy as jnp
from jax import lax
from jax.experimental import pallas as pl
from jax.experimental.pallas import tpu as pltpu
from jax.experimental.pallas import tpu_sc as plsc

B = 16384
DIM = 32
NROWS = 1000000
NC = 2
NS = 16
NW = NC * NS
PW = B // NW
SLAB = 896
NPAD = ((NROWS + 127) // 128) * 128
MAXBASE = NPAD - SLAB


def _count_below(idxv, end):
    n = jnp.int32(0)
    for g in range(PW // 16):
        v = idxv[g]
        n = n + jnp.sum((v < end).astype(jnp.int32))
    return n


def _id_at(idxv, p, lanes):
    v = idxv[p // 16]
    return jnp.sum(v * (lanes == p % 16).astype(jnp.int32))


def _sc_scan_body(ids_hbm, pos_hbm, tt, out, idxv, pos_v, ring, buf,
                  sem0, sem1, sem_st):
    wid = lax.axis_index("s") * NC + lax.axis_index("c")
    pltpu.sync_copy(ids_hbm.at[pl.ds(wid * (PW // 16), PW // 16)], idxv)
    pltpu.sync_copy(pos_hbm.at[pl.ds(wid * (PW // 128), PW // 128)], pos_v)

    lanes = lax.iota(jnp.int32, 16)
    first = _id_at(idxv, jnp.int32(0), lanes)
    last = _id_at(idxv, jnp.int32(PW - 1), lanes)
    c_lo = jnp.minimum((first // 128) * 128, MAXBASE)
    n_slabs = (last - c_lo) // SLAB + 1
    n_pairs = (n_slabs + 1) // 2
    r0 = lax.iota(jnp.int32, 16)
    r1 = r0 + 16

    def slab_base(s):
        return pl.multiple_of(jnp.minimum(c_lo + s * SLAB, MAXBASE), 128)

    def start(s, slot, sem):
        pltpu.make_async_copy(
            tt.at[:, pl.ds(slab_base(s), SLAB)], ring.at[slot], sem).start()

    def wait(slot, sem):
        pltpu.make_async_copy(
            tt.at[:, pl.ds(0, SLAB)], ring.at[slot], sem).wait()

    def extract(s, slot, ptr):
        base = slab_base(s)
        nend = _count_below(idxv, base + SLAB)

        def ext(p, c):
            col = _id_at(idxv, p, lanes) - base
            cv = jnp.full((16,), col, jnp.int32)
            v0 = plsc.load_gather(ring.at[slot], [r0, cv])
            v1 = plsc.load_gather(ring.at[slot], [r1, cv])
            buf[p, pl.ds(0, 16)] = v0
            buf[p, pl.ds(16, 16)] = v1
            return c

        lax.fori_loop(ptr, nend, ext, jnp.int32(0))
        return nend

    start(jnp.int32(0), 0, sem0)

    def pair_step(k, ptr):
        s0 = 2 * k
        start(s0 + 1, 1, sem1)
        wait(0, sem0)
        ptr = extract(s0, 0, ptr)
        start(s0 + 2, 0, sem0)
        wait(1, sem1)
        return extract(s0 + 1, 1, ptr)

    lax.fori_loop(0, n_pairs, pair_step, jnp.int32(0))
    wait(0, sem0)

    sc = []
    for j in range(PW // 128):
        sc.append(pltpu.async_copy(
            buf.at[pl.ds(j * 128, 128)], out.at[pos_v.at[j]], sem_st))
    for c in sc:
        c.wait()


_sc_scan = functools.partial(
    pl.kernel,
    out_type=jax.ShapeDtypeStruct((B, 128), jnp.float32),
    mesh=plsc.VectorSubcoreMesh(core_axis_name="c", subcore_axis_name="s"),
    scratch_types=[
        pltpu.VMEM((PW // 16, 16), jnp.int32),
        pltpu.VMEM((PW // 128, 128), jnp.int32),
        pltpu.VMEM((2, DIM, SLAB), jnp.float32),
        pltpu.VMEM((PW, 128), jnp.float32),
        pltpu.SemaphoreType.DMA,
        pltpu.SemaphoreType.DMA,
        pltpu.SemaphoreType.DMA,
    ],
    compiler_params=pltpu.CompilerParams(needs_layout_passes=False),
)(_sc_scan_body)


def _tc_dense_body(gu_ref, gi_ref, wu_ref, bu_ref, wi_ref, bi_ref, out_ref):
    ue = gu_ref[...][:, :DIM]
    ie = gi_ref[...][:, :DIM]
    u = jnp.maximum(
        jnp.dot(ue, wu_ref[...], preferred_element_type=jnp.float32)
        + bu_ref[...], 0.0)
    v = jnp.maximum(
        jnp.dot(ie, wi_ref[...], preferred_element_type=jnp.float32)
        + bi_ref[...], 0.0)
    out_ref[...] = jnp.sum(u * v, axis=1, keepdims=True)


def _tc_dense(gu, gi, Wu, bu2, Wi, bi2):
    blk = 2048
    grid = B // blk
    return pl.pallas_call(
        _tc_dense_body,
        grid=(grid,),
        in_specs=[
            pl.BlockSpec((blk, 128), lambda i: (i, 0)),
            pl.BlockSpec((blk, 128), lambda i: (i, 0)),
            pl.BlockSpec((DIM, DIM), lambda i: (0, 0)),
            pl.BlockSpec((1, DIM), lambda i: (0, 0)),
            pl.BlockSpec((DIM, DIM), lambda i: (0, 0)),
            pl.BlockSpec((1, DIM), lambda i: (0, 0)),
        ],
        out_specs=pl.BlockSpec((blk, 1), lambda i: (i, 0)),
        out_shape=jax.ShapeDtypeStruct((B, 1), jnp.float32),
    )(gu, gi, Wu, bu2, Wi, bi2)


def kernel(user_table, item_table, Wu, bu, Wi, bi, user_ids, item_ids):
    order_u = jnp.argsort(user_ids, stable=False).astype(jnp.int32)
    order_i = jnp.argsort(item_ids, stable=False).astype(jnp.int32)
    su = user_ids[order_u].reshape(B // 16, 16)
    si = item_ids[order_i].reshape(B // 16, 16)
    pu_pos = order_u.reshape(B // 128, 128)
    pi_pos = order_i.reshape(B // 128, 128)
    gu = _sc_scan(su, pu_pos, user_table.T)
    gi = _sc_scan(si, pi_pos, item_table.T)

    scores = _tc_dense(gu, gi, Wu, bu.reshape(1, DIM), Wi, bi.reshape(1, DIM))
    return scores.reshape(B)
